# baseline (speedup 1.0000x reference)
import functools

import jax
import jax.numpy as jnp
import numpy as np
from jax import lax
from jax.experimental import pallas as pl
from jax.experimental.pallas import tpu as pltpu
from jax.experimental.pallas import tpu_sc as plsc

N = 50000
E = 1600000
HID = 256
NW = 32
NPW = 1568
NPAD = NW * NPW
C = 3072
CR = C // 128
G = C // 16
EPAD = E + C
ER = EPAD // 128
BIG = 3.0e38
BM = 1000

_PERM = np.array([0, 1, 2, 12, 13, 14, 24,
                  3, 4, 5, 15, 16, 17, 25,
                  6, 7, 8, 18, 19, 20, 26,
                  9, 10, 11, 21, 22, 23, 27], dtype=np.int32)


def _permute(x, idx):
    return jnp.take_along_axis(x, idx, axis=0, mode="promise_in_bounds")


def _gather_descs(table_hbm, sidx_v, rows_v, sem):
    return [pltpu.make_async_copy(table_hbm.at[sidx_v.at[j]],
                                  rows_v.at[pl.ds(j * 128, 128)], sem)
            for j in range(CR)]


def _sc_body(qidx_hbm, sidx_hbm, table_hbm, bounds_hbm,
             out_hbm,
             bounds_v, qidx0, qidx1, sidx0, sidx1, rows0, rows1, acc_v,
             semq0, semq1, sems0, sems1, semr0, semr1):
    cid = lax.axis_index("c")
    sid = lax.axis_index("s")
    wid = sid * 2 + cid

    pltpu.sync_copy(bounds_hbm, bounds_v)
    iota = lax.iota(jnp.int32, 16)

    b0 = bounds_v[pl.ds(0, 16)]
    b1 = bounds_v[pl.ds(16, 16)]
    b2 = bounds_v[pl.ds(32, 16)]

    def bound(j):
        w = j // 16
        l = j % 16
        row = jnp.where(w == 0, b0, jnp.where(w == 1, b1, b2))
        bcast = _permute(row, jnp.full((16,), l, jnp.int32))
        return bcast[0]

    e0 = bound(wid)
    e1 = bound(wid + 1)
    n0 = pl.multiple_of(wid * NPW, 8)
    n1 = jnp.minimum(n0 + NPW, N)

    zeros16 = jnp.zeros((16,), jnp.float32)
    mmpat = jnp.where(iota < 7, BIG, jnp.where(iota < 14, -BIG, 0.0))

    def init_body(i, carry):
        acc_v[i, pl.ds(0, 16)] = zeros16
        acc_v[i, pl.ds(16, 16)] = mmpat
        return carry

    lax.fori_loop(0, NPW, init_body, 0)

    e0a = (e0 // C) * C
    nchunks = (e1 - e0a + C - 1) // C

    idx_up = jnp.minimum(iota + 1, 15)
    ks = (1, 2, 4, 8)
    idx_dn = [jnp.maximum(iota - k, 0) for k in ks]
    iota_f = iota.astype(jnp.float32)
    cols = [jnp.full((16,), f, jnp.int32) for f in range(16)]

    qidx_b = (qidx0, qidx1)
    sidx_b = (sidx0, sidx1)
    rows_b = (rows0, rows1)
    semq = (semq0, semq1)
    sems = (sems0, sems1)
    semr = (semr0, semr1)

    def qidx_desc(j, p):
        base = e0a + j * C
        rb = pl.multiple_of(base // 128, 8)
        return pltpu.make_async_copy(qidx_hbm.at[pl.ds(rb, CR)],
                                     qidx_b[p], semq[p])

    def sidx_desc(j, p):
        base = e0a + j * C
        rb = pl.multiple_of(base // 128, 8)
        return pltpu.make_async_copy(sidx_hbm.at[pl.ds(rb, CR)],
                                     sidx_b[p], sems[p])

    def fire_gathers(p):
        for d in _gather_descs(table_hbm, sidx_b[p], rows_b[p], semr[p]):
            d.start()

    def wait_gathers(p):
        for d in _gather_descs(table_hbm, sidx_b[p], rows_b[p], semr[p]):
            d.wait()

    def compute_chunk(ci, p, qidx_v, rows_v):
        def group_body(g, gcarry):
            r = g // 8
            o = (g % 8) * 16
            seg = qidx_v[r, pl.ds(o, 16)]
            valid = (seg >= n0) & (seg < n1)
            qloc = jnp.clip(seg - n0, 0, NPW - 1)

            eidx = g * 16 + iota
            sf = [plsc.load_gather(rows_v, [eidx, cols[f]])
                  for f in range(7)]
            cs = [plsc.cumsum(v) for v in sf]
            cs2 = [plsc.cumsum(v * v) for v in sf]

            def fast_path():
                m15 = (iota == 15) & valid
                for f in range(7):
                    plsc.addupdate_scatter(acc_v, [qloc, cols[f]], cs[f],
                                           mask=m15)
                    plsc.addupdate_scatter(acc_v, [qloc, cols[7 + f]],
                                           cs2[f], mask=m15)
                    mx = plsc.cummax(sf[f])
                    cur2 = plsc.load_gather(acc_v, [qloc, cols[7 + f] + 16],
                                            mask=m15)
                    plsc.store_scatter(acc_v, [qloc, cols[7 + f] + 16],
                                       jnp.maximum(cur2, mx), mask=m15)
                    mn = -plsc.cummax(-sf[f])
                    cur = plsc.load_gather(acc_v, [qloc, cols[f] + 16],
                                           mask=m15)
                    plsc.store_scatter(acc_v, [qloc, cols[f] + 16],
                                       jnp.minimum(cur, mn), mask=m15)
                plsc.addupdate_scatter(acc_v, [qloc, cols[14]],
                                       jnp.full((16,), 16.0, jnp.float32),
                                       mask=m15)

            def slow_path():
                sprev = _permute(seg, idx_dn[0])
                snext = _permute(seg, idx_up)
                last = (seg != snext) | (iota == 15)
                first = (seg != sprev) | (iota == 0)
                m_last = last & valid
                m_first = first & valid & (iota > 0)

                ms = [(iota >= 1) & (sprev == seg)]
                for k, idn in zip(ks[1:], idx_dn[1:]):
                    segk = _permute(seg, idn)
                    ms.append((iota >= k) & (segk == seg))

                def segscan(v, op):
                    for idn, m in zip(idx_dn, ms):
                        vk = _permute(v, idn)
                        v = jnp.where(m, op(v, vk), v)
                    return v

                for f in range(7):
                    cp = _permute(cs[f], idx_dn[0])
                    plsc.addupdate_scatter(acc_v, [qloc, cols[f]], cs[f],
                                           mask=m_last)
                    plsc.addupdate_scatter(acc_v, [qloc, cols[f]], -cp,
                                           mask=m_first)
                    cp2 = _permute(cs2[f], idx_dn[0])
                    plsc.addupdate_scatter(acc_v, [qloc, cols[7 + f]],
                                           cs2[f], mask=m_last)
                    plsc.addupdate_scatter(acc_v, [qloc, cols[7 + f]], -cp2,
                                           mask=m_first)
                    mn = segscan(sf[f], jnp.minimum)
                    cur = plsc.load_gather(acc_v, [qloc, cols[f] + 16],
                                           mask=m_last)
                    plsc.store_scatter(acc_v, [qloc, cols[f] + 16],
                                       jnp.minimum(cur, mn), mask=m_last)
                    mx = segscan(sf[f], jnp.maximum)
                    cur2 = plsc.load_gather(acc_v, [qloc, cols[7 + f] + 16],
                                            mask=m_last)
                    plsc.store_scatter(acc_v, [qloc, cols[7 + f] + 16],
                                       jnp.maximum(cur2, mx), mask=m_last)

                plsc.addupdate_scatter(acc_v, [qloc, cols[14]],
                                       iota_f + 1.0, mask=m_last)
                plsc.addupdate_scatter(acc_v, [qloc, cols[14]], -iota_f,
                                       mask=m_first)

            lax.cond(seg[0] == seg[15], fast_path, slow_path)
            return gcarry

        lax.fori_loop(0, G, group_body, 0)

    @pl.when(nchunks > 0)
    def _():
        qidx_desc(0, 0).start()
        sidx_desc(0, 0).start()
        sidx_desc(0, 0).wait()
        fire_gathers(0)

    @pl.when(nchunks > 1)
    def _():
        qidx_desc(1, 1).start()
        sidx_desc(1, 1).start()

    def pipe_body(t, carry):
        j0 = t * 2
        for p in (0, 1):
            j = j0 + p

            @pl.when(j < nchunks)
            def _(j=j, p=p):
                wait_gathers(p)

                @pl.when(j + 1 < nchunks)
                def _():
                    sidx_desc(j + 1, 1 - p).wait()
                    fire_gathers(1 - p)

                @pl.when(j + 2 < nchunks)
                def _():
                    sidx_desc(j + 2, p).start()

                qidx_desc(j, p).wait()
                compute_chunk(j, p, qidx_b[p], rows_b[p])

                @pl.when(j + 2 < nchunks)
                def _():
                    qidx_desc(j + 2, p).start()
        return carry

    lax.fori_loop(0, (nchunks + 1) // 2, pipe_body, 0)

    pltpu.sync_copy(acc_v, out_hbm.at[pl.ds(n0, NPW)])


def _gelu(x):
    return 0.5 * x * (1.0 + lax.erf(x * 0.7071067811865476))


def _mlp_body(x_ref, q_ref, w1_ref, b1_ref, w2_ref, b2_ref, o_ref):
    x = x_ref[...]
    q = q_ref[...]
    cnt_raw = x[:, 14:15]
    cnt = jnp.maximum(cnt_raw, 1.0)
    mean_s = x[:, 0:7] / cnt
    ex2 = x[:, 7:14] / cnt
    std = jnp.sqrt(jnp.maximum(ex2 - mean_s * mean_s, 0.0))
    qpos = q[:, 0:3]
    qeff = jnp.where(cnt_raw > 0.5, qpos, 0.0)
    mean = jnp.concatenate([mean_s[:, 0:3] - qeff, mean_s[:, 3:7]], axis=-1)
    mn_s = jnp.concatenate([x[:, 16:19] - qpos, x[:, 19:23]], axis=-1)
    mx_s = jnp.concatenate([x[:, 23:26] - qpos, x[:, 26:30]], axis=-1)
    mn = jnp.clip(mn_s, -100.0, 100.0)
    mx = jnp.clip(mx_s, -100.0, 100.0)
    feat = jnp.concatenate(
        [mean, std, mn, mx, jnp.zeros((x.shape[0], 4), jnp.float32)], axis=-1)
    h = jnp.dot(feat, w1_ref[...], preferred_element_type=jnp.float32)
    h = _gelu(h + b1_ref[...])
    h = jnp.dot(h, w2_ref[...], preferred_element_type=jnp.float32)
    o_ref[...] = _gelu(h + b2_ref[...])


@jax.jit
def kernel(query_pos, support_pos, q_idx, s_idx, support_normals,
           support_curvature, W1, b1, W2, b2):
    table = jnp.concatenate(
        [support_pos, support_normals, support_curvature,
         jnp.zeros((N, 1), jnp.float32)], axis=1)
    qpos4 = jnp.zeros((NPAD, 4), jnp.float32).at[:N, :3].set(query_pos)
    bounds = jnp.searchsorted(
        q_idx, jnp.arange(NW + 1, dtype=jnp.int32) * NPW).astype(jnp.int32)
    bounds = jnp.concatenate(
        [bounds, jnp.full((48 - (NW + 1),), E, jnp.int32)])
    qidx2d = jnp.concatenate(
        [q_idx, jnp.full((EPAD - E,), N, jnp.int32)]).reshape(ER, 128)
    sidx2d = jnp.concatenate(
        [s_idx, jnp.zeros((EPAD - E,), jnp.int32)]).reshape(ER, 128)

    mesh = plsc.VectorSubcoreMesh(core_axis_name="c", subcore_axis_name="s",
                                  num_cores=2, num_subcores=16)
    raw = pl.kernel(
        _sc_body,
        out_type=jax.ShapeDtypeStruct((NPAD, 32), jnp.float32),
        mesh=mesh,
        scratch_types=[
            pltpu.VMEM((48,), jnp.int32),
            pltpu.VMEM((CR, 128), jnp.int32),
            pltpu.VMEM((CR, 128), jnp.int32),
            pltpu.VMEM((CR, 128), jnp.int32),
            pltpu.VMEM((CR, 128), jnp.int32),
            pltpu.VMEM((C, 8), jnp.float32),
            pltpu.VMEM((C, 8), jnp.float32),
            pltpu.VMEM((NPW, 32), jnp.float32),
            pltpu.SemaphoreType.DMA,
            pltpu.SemaphoreType.DMA,
            pltpu.SemaphoreType.DMA,
            pltpu.SemaphoreType.DMA,
            pltpu.SemaphoreType.DMA,
            pltpu.SemaphoreType.DMA,
        ],
        compiler_params=pltpu.CompilerParams(needs_layout_passes=False,
                                             use_tc_tiling_on_sc=False),
    )(qidx2d, sidx2d, table, bounds)

    W1r = jnp.concatenate(
        [W1[_PERM], jnp.zeros((4, HID), jnp.float32)], axis=0)

    out = pl.pallas_call(
        _mlp_body,
        grid=(N // BM,),
        in_specs=[
            pl.BlockSpec((BM, 32), lambda i: (i, 0)),
            pl.BlockSpec((BM, 4), lambda i: (i, 0)),
            pl.BlockSpec((32, HID), lambda i: (0, 0)),
            pl.BlockSpec((1, HID), lambda i: (0, 0)),
            pl.BlockSpec((HID, HID), lambda i: (0, 0)),
            pl.BlockSpec((1, HID), lambda i: (0, 0)),
        ],
        out_specs=pl.BlockSpec((BM, HID), lambda i: (i, 0)),
        out_shape=jax.ShapeDtypeStruct((N, HID), jnp.float32),
    )(raw, qpos4, W1r, b1.reshape(1, HID), W2, b2.reshape(1, HID))

    return out


# speedup vs baseline: 1.0247x; 1.0247x over previous
import functools

import jax
import jax.numpy as jnp
import numpy as np
from jax import lax
from jax.experimental import pallas as pl
from jax.experimental.pallas import tpu as pltpu
from jax.experimental.pallas import tpu_sc as plsc

N = 50000
E = 1600000
HID = 256
NW = 32
NPW = 1568
NPAD = NW * NPW
C = 2048
CR = C // 128
G = C // 16
EPAD = E + C
ER = EPAD // 128
BIG = 3.0e38
BM = 1000

_PERM = np.array([0, 1, 2, 12, 13, 14, 24,
                  3, 4, 5, 15, 16, 17, 25,
                  6, 7, 8, 18, 19, 20, 26,
                  9, 10, 11, 21, 22, 23, 27], dtype=np.int32)


def _permute(x, idx):
    return jnp.take_along_axis(x, idx, axis=0, mode="promise_in_bounds")


def _gather_descs(table_hbm, sidx_v, rows_v, sem):
    return [pltpu.make_async_copy(table_hbm.at[sidx_v.at[j]],
                                  rows_v.at[pl.ds(j * 128, 128)], sem)
            for j in range(CR)]


def _sc_body(qidx_hbm, sidx_hbm, table_hbm, bounds_hbm,
             out_hbm,
             bounds_v, qidx0, qidx1, sidx0, sidx1, rows0, rows1, acc_v,
             semq0, semq1, sems0, sems1, semr0, semr1):
    cid = lax.axis_index("c")
    sid = lax.axis_index("s")
    wid = sid * 2 + cid

    pltpu.sync_copy(bounds_hbm, bounds_v)
    iota = lax.iota(jnp.int32, 16)

    b0 = bounds_v[pl.ds(0, 16)]
    b1 = bounds_v[pl.ds(16, 16)]
    b2 = bounds_v[pl.ds(32, 16)]

    def bound(j):
        w = j // 16
        l = j % 16
        row = jnp.where(w == 0, b0, jnp.where(w == 1, b1, b2))
        bcast = _permute(row, jnp.full((16,), l, jnp.int32))
        return bcast[0]

    e0 = bound(wid)
    e1 = bound(wid + 1)
    n0 = pl.multiple_of(wid * NPW, 8)
    n1 = jnp.minimum(n0 + NPW, N)

    zeros16 = jnp.zeros((16,), jnp.float32)
    mmpat = jnp.where(iota < 7, BIG, jnp.where(iota < 14, -BIG, 0.0))

    def init_body(i, carry):
        acc_v[i, pl.ds(0, 16)] = zeros16
        acc_v[i, pl.ds(16, 16)] = mmpat
        return carry

    lax.fori_loop(0, NPW, init_body, 0)

    e0a = (e0 // C) * C
    nchunks = (e1 - e0a + C - 1) // C

    idx_up = jnp.minimum(iota + 1, 15)
    ks = (1, 2, 4, 8)
    idx_dn = [jnp.maximum(iota - k, 0) for k in ks]
    iota_f = iota.astype(jnp.float32)
    cols = [jnp.full((16,), f, jnp.int32) for f in range(16)]

    qidx_b = (qidx0, qidx1)
    sidx_b = (sidx0, sidx1)
    rows_b = (rows0, rows1)
    semq = (semq0, semq1)
    sems = (sems0, sems1)
    semr = (semr0, semr1)

    def qidx_desc(j, p):
        base = e0a + j * C
        rb = pl.multiple_of(base // 128, 8)
        return pltpu.make_async_copy(qidx_hbm.at[pl.ds(rb, CR)],
                                     qidx_b[p], semq[p])

    def sidx_desc(j, p):
        base = e0a + j * C
        rb = pl.multiple_of(base // 128, 8)
        return pltpu.make_async_copy(sidx_hbm.at[pl.ds(rb, CR)],
                                     sidx_b[p], sems[p])

    def fire_gathers(p):
        for d in _gather_descs(table_hbm, sidx_b[p], rows_b[p], semr[p]):
            d.start()

    def wait_gathers(p):
        for d in _gather_descs(table_hbm, sidx_b[p], rows_b[p], semr[p]):
            d.wait()

    def compute_chunk(ci, p, qidx_v, rows_v):
        def group_body(g, gcarry):
            r = g // 8
            o = (g % 8) * 16
            seg = qidx_v[r, pl.ds(o, 16)]
            valid = (seg >= n0) & (seg < n1)
            qloc = jnp.clip(seg - n0, 0, NPW - 1)

            eidx = g * 16 + iota
            sf = [plsc.load_gather(rows_v, [eidx, cols[f]])
                  for f in range(7)]
            cs = [plsc.cumsum(v) for v in sf]
            cs2 = [plsc.cumsum(v * v) for v in sf]

            def fast_path():
                m15 = (iota == 15) & valid
                for f in range(7):
                    plsc.addupdate_scatter(acc_v, [qloc, cols[f]], cs[f],
                                           mask=m15)
                    plsc.addupdate_scatter(acc_v, [qloc, cols[7 + f]],
                                           cs2[f], mask=m15)
                    mx = plsc.cummax(sf[f])
                    cur2 = plsc.load_gather(acc_v, [qloc, cols[7 + f] + 16],
                                            mask=m15)
                    plsc.store_scatter(acc_v, [qloc, cols[7 + f] + 16],
                                       jnp.maximum(cur2, mx), mask=m15)
                    mn = -plsc.cummax(-sf[f])
                    cur = plsc.load_gather(acc_v, [qloc, cols[f] + 16],
                                           mask=m15)
                    plsc.store_scatter(acc_v, [qloc, cols[f] + 16],
                                       jnp.minimum(cur, mn), mask=m15)
                plsc.addupdate_scatter(acc_v, [qloc, cols[14]],
                                       jnp.full((16,), 16.0, jnp.float32),
                                       mask=m15)

            def slow_path():
                sprev = _permute(seg, idx_dn[0])
                snext = _permute(seg, idx_up)
                last = (seg != snext) | (iota == 15)
                first = (seg != sprev) | (iota == 0)
                m_last = last & valid
                m_first = first & valid & (iota > 0)

                ms = [(iota >= 1) & (sprev == seg)]
                for k, idn in zip(ks[1:], idx_dn[1:]):
                    segk = _permute(seg, idn)
                    ms.append((iota >= k) & (segk == seg))

                def segscan(v, op):
                    for idn, m in zip(idx_dn, ms):
                        vk = _permute(v, idn)
                        v = jnp.where(m, op(v, vk), v)
                    return v

                for f in range(7):
                    cp = _permute(cs[f], idx_dn[0])
                    plsc.addupdate_scatter(acc_v, [qloc, cols[f]], cs[f],
                                           mask=m_last)
                    plsc.addupdate_scatter(acc_v, [qloc, cols[f]], -cp,
                                           mask=m_first)
                    cp2 = _permute(cs2[f], idx_dn[0])
                    plsc.addupdate_scatter(acc_v, [qloc, cols[7 + f]],
                                           cs2[f], mask=m_last)
                    plsc.addupdate_scatter(acc_v, [qloc, cols[7 + f]], -cp2,
                                           mask=m_first)
                    mn = segscan(sf[f], jnp.minimum)
                    cur = plsc.load_gather(acc_v, [qloc, cols[f] + 16],
                                           mask=m_last)
                    plsc.store_scatter(acc_v, [qloc, cols[f] + 16],
                                       jnp.minimum(cur, mn), mask=m_last)
                    mx = segscan(sf[f], jnp.maximum)
                    cur2 = plsc.load_gather(acc_v, [qloc, cols[7 + f] + 16],
                                            mask=m_last)
                    plsc.store_scatter(acc_v, [qloc, cols[7 + f] + 16],
                                       jnp.maximum(cur2, mx), mask=m_last)

                plsc.addupdate_scatter(acc_v, [qloc, cols[14]],
                                       iota_f + 1.0, mask=m_last)
                plsc.addupdate_scatter(acc_v, [qloc, cols[14]], -iota_f,
                                       mask=m_first)

            lax.cond(seg[0] == seg[15], fast_path, slow_path)
            return gcarry

        lax.fori_loop(0, G, group_body, 0)

    @pl.when(nchunks > 0)
    def _():
        qidx_desc(0, 0).start()
        sidx_desc(0, 0).start()
        sidx_desc(0, 0).wait()
        fire_gathers(0)

    @pl.when(nchunks > 1)
    def _():
        qidx_desc(1, 1).start()
        sidx_desc(1, 1).start()

    def pipe_body(t, carry):
        j0 = t * 2
        for p in (0, 1):
            j = j0 + p

            @pl.when(j < nchunks)
            def _(j=j, p=p):
                wait_gathers(p)

                @pl.when(j + 1 < nchunks)
                def _():
                    sidx_desc(j + 1, 1 - p).wait()
                    fire_gathers(1 - p)

                @pl.when(j + 2 < nchunks)
                def _():
                    sidx_desc(j + 2, p).start()

                qidx_desc(j, p).wait()
                compute_chunk(j, p, qidx_b[p], rows_b[p])

                @pl.when(j + 2 < nchunks)
                def _():
                    qidx_desc(j + 2, p).start()
        return carry

    lax.fori_loop(0, (nchunks + 1) // 2, pipe_body, 0)

    pltpu.sync_copy(acc_v, out_hbm.at[pl.ds(n0, NPW)])


def _gelu(x):
    return 0.5 * x * (1.0 + lax.erf(x * 0.7071067811865476))


def _mlp_body(x_ref, q_ref, w1_ref, b1_ref, w2_ref, b2_ref, o_ref):
    x = x_ref[...]
    q = q_ref[...]
    cnt_raw = x[:, 14:15]
    cnt = jnp.maximum(cnt_raw, 1.0)
    mean_s = x[:, 0:7] / cnt
    ex2 = x[:, 7:14] / cnt
    std = jnp.sqrt(jnp.maximum(ex2 - mean_s * mean_s, 0.0))
    qpos = q[:, 0:3]
    qeff = jnp.where(cnt_raw > 0.5, qpos, 0.0)
    mean = jnp.concatenate([mean_s[:, 0:3] - qeff, mean_s[:, 3:7]], axis=-1)
    mn_s = jnp.concatenate([x[:, 16:19] - qpos, x[:, 19:23]], axis=-1)
    mx_s = jnp.concatenate([x[:, 23:26] - qpos, x[:, 26:30]], axis=-1)
    mn = jnp.clip(mn_s, -100.0, 100.0)
    mx = jnp.clip(mx_s, -100.0, 100.0)
    feat = jnp.concatenate(
        [mean, std, mn, mx, jnp.zeros((x.shape[0], 4), jnp.float32)], axis=-1)
    h = jnp.dot(feat, w1_ref[...], preferred_element_type=jnp.float32)
    h = _gelu(h + b1_ref[...])
    h = jnp.dot(h, w2_ref[...], preferred_element_type=jnp.float32)
    o_ref[...] = _gelu(h + b2_ref[...])


@jax.jit
def kernel(query_pos, support_pos, q_idx, s_idx, support_normals,
           support_curvature, W1, b1, W2, b2):
    table = jnp.concatenate(
        [support_pos, support_normals, support_curvature,
         jnp.zeros((N, 1), jnp.float32)], axis=1)
    qpos4 = jnp.zeros((NPAD, 4), jnp.float32).at[:N, :3].set(query_pos)
    bounds = jnp.searchsorted(
        q_idx, jnp.arange(NW + 1, dtype=jnp.int32) * NPW).astype(jnp.int32)
    bounds = jnp.concatenate(
        [bounds, jnp.full((48 - (NW + 1),), E, jnp.int32)])
    qidx2d = jnp.concatenate(
        [q_idx, jnp.full((EPAD - E,), N, jnp.int32)]).reshape(ER, 128)
    sidx2d = jnp.concatenate(
        [s_idx, jnp.zeros((EPAD - E,), jnp.int32)]).reshape(ER, 128)

    mesh = plsc.VectorSubcoreMesh(core_axis_name="c", subcore_axis_name="s",
                                  num_cores=2, num_subcores=16)
    raw = pl.kernel(
        _sc_body,
        out_type=jax.ShapeDtypeStruct((NPAD, 32), jnp.float32),
        mesh=mesh,
        scratch_types=[
            pltpu.VMEM((48,), jnp.int32),
            pltpu.VMEM((CR, 128), jnp.int32),
            pltpu.VMEM((CR, 128), jnp.int32),
            pltpu.VMEM((CR, 128), jnp.int32),
            pltpu.VMEM((CR, 128), jnp.int32),
            pltpu.VMEM((C, 8), jnp.float32),
            pltpu.VMEM((C, 8), jnp.float32),
            pltpu.VMEM((NPW, 32), jnp.float32),
            pltpu.SemaphoreType.DMA,
            pltpu.SemaphoreType.DMA,
            pltpu.SemaphoreType.DMA,
            pltpu.SemaphoreType.DMA,
            pltpu.SemaphoreType.DMA,
            pltpu.SemaphoreType.DMA,
        ],
        compiler_params=pltpu.CompilerParams(needs_layout_passes=False,
                                             use_tc_tiling_on_sc=False),
    )(qidx2d, sidx2d, table, bounds)

    W1r = jnp.concatenate(
        [W1[_PERM], jnp.zeros((4, HID), jnp.float32)], axis=0)

    out = pl.pallas_call(
        _mlp_body,
        grid=(N // BM,),
        in_specs=[
            pl.BlockSpec((BM, 32), lambda i: (i, 0)),
            pl.BlockSpec((BM, 4), lambda i: (i, 0)),
            pl.BlockSpec((32, HID), lambda i: (0, 0)),
            pl.BlockSpec((1, HID), lambda i: (0, 0)),
            pl.BlockSpec((HID, HID), lambda i: (0, 0)),
            pl.BlockSpec((1, HID), lambda i: (0, 0)),
        ],
        out_specs=pl.BlockSpec((BM, HID), lambda i: (i, 0)),
        out_shape=jax.ShapeDtypeStruct((N, HID), jnp.float32),
    )(raw, qpos4, W1r, b1.reshape(1, HID), W2, b2.reshape(1, HID))

    return out
